# baseline (speedup 1.0000x reference)
import functools

import jax
import jax.numpy as jnp
from jax import lax
from jax.experimental import pallas as pl
from jax.experimental.pallas import tpu as pltpu
from jax.experimental.pallas import tpu_sc as plsc

N = 50000
D = 256
S = 512
NEG = -1e38

NR = 16
CHUNK = 3136
NPAD = NR * CHUNK
SUB = 112
NSUB = CHUNK // SUB
VECS = CHUNK // 16
DH = D // 2

BCH = NPAD // 32
BVECS = BCH // 16


BLK = 1568
NBLK = NPAD // BLK


def _gate_kernel(x_ref, w1_ref, b1_ref, w2_ref, b2_ref, gate_ref,
                 xl_ref, xr_ref):
  b = pl.program_id(0)
  x = x_ref[...]
  h = jnp.maximum(x @ w1_ref[...] + b1_ref[...], 0.0)
  gate = h @ w2_ref[...] + b2_ref[...]
  row = b * BLK + lax.broadcasted_iota(jnp.int32, (BLK, 1), 0)
  valid = row < N
  gate_ref[...] = jnp.where(valid, gate, NEG)
  xz = jnp.where(valid, x, 0.0).astype(jnp.bfloat16)
  xl_ref[...] = xz[:, 0:DH]
  xr_ref[...] = xz[:, DH:D]


def _gate(x, W1, b1, W2, b2):
  return pl.pallas_call(
      _gate_kernel,
      grid=(NBLK,),
      in_specs=[
          pl.BlockSpec((BLK, D), lambda b: (b, 0)),
          pl.BlockSpec((D, D), lambda b: (0, 0)),
          pl.BlockSpec((1, D), lambda b: (0, 0)),
          pl.BlockSpec((D, 1), lambda b: (0, 0)),
          pl.BlockSpec((1, 1), lambda b: (0, 0)),
      ],
      out_specs=[
          pl.BlockSpec((BLK, 1), lambda b: (b, 0)),
          pl.BlockSpec((BLK, DH), lambda b: (b, 0)),
          pl.BlockSpec((BLK, DH), lambda b: (b, 0)),
      ],
      out_shape=[
          jax.ShapeDtypeStruct((NPAD, 1), jnp.float32),
          jax.ShapeDtypeStruct((NPAD, DH), jnp.bfloat16),
          jax.ShapeDtypeStruct((NPAD, DH), jnp.bfloat16),
      ],
  )(x, W1, b1.reshape(1, D), W2, b2.reshape(1, 1))


def _pool_kernel(xl_hbm, xr_hbm, gate_hbm, seg_hbm,
                 part_hbm, partd_hbm, partm_hbm,
                 gate_v, seg_v, wv, segmax_v, tab, dtab, dsum,
                 xbuf0, xbuf1, xbuf2, xbuf3, acc,
                 gsem0, gsem1, gsem2, gsem3):
  cid = lax.axis_index("c")
  sid = lax.axis_index("s")
  base = sid * CHUNK
  li = lax.iota(jnp.int32, 16)

  def gather(j, xb, sem):
    off = (base + j * SUB) * DH

    @pl.when(cid == 0)
    def _():
      pltpu.async_copy(xl_hbm.at[pl.ds(off, SUB * DH)], xb, sem)

    @pl.when(cid == 1)
    def _():
      pltpu.async_copy(xr_hbm.at[pl.ds(off, SUB * DH)], xb, sem)

  def drain(xb, sem):
    pltpu.make_async_copy(xl_hbm.at[pl.ds(0, SUB * DH)], xb, sem).wait()

  xbufs = (xbuf0, xbuf1, xbuf2, xbuf3)
  gsems = (gsem0, gsem1, gsem2, gsem3)

  for b in range(4):
    gather(b, xbufs[b], gsems[b])

  @plsc.parallel_loop(0, S, unroll=4)
  def _(r):
    for c in range(DH // 16):
      acc[r, pl.ds(c * 16, 16)] = jnp.zeros((16,), jnp.float32)

  @plsc.parallel_loop(0, 16, unroll=2)
  def _(l):
    for k in range(S // 16):
      dtab[l, pl.ds(k * 16, 16)] = jnp.zeros((16,), jnp.float32)
      tab[l, pl.ds(k * 16, 16)] = jnp.full((16,), NEG, jnp.float32)

  pltpu.sync_copy(gate_hbm.at[sid], gate_v)

  tail = N - (NR - 1) * CHUNK

  @pl.when(sid < NR - 1)
  def _():
    pltpu.sync_copy(seg_hbm.at[pl.ds(base, CHUNK)], seg_v)

  @pl.when(sid == NR - 1)
  def _():
    pltpu.sync_copy(seg_hbm.at[pl.ds(base, tail)], seg_v.at[pl.ds(0, tail)])
    for t in range((CHUNK - tail) // 16):
      seg_v[pl.ds(tail + t * 16, 16)] = jnp.zeros((16,), jnp.int32)

  def maxbody(i, _):
    g = gate_v[pl.ds(i * 16, 16)]
    sg = seg_v[pl.ds(i * 16, 16)]
    cur = plsc.load_gather(tab, [li, sg])
    plsc.store_scatter(tab, [li, sg], jnp.maximum(cur, g))
    return 0
  lax.fori_loop(0, VECS, maxbody, 0)

  def mfold(k, _):
    m = tab[0, pl.ds(k * 16, 16)]
    for l in range(1, 16):
      m = jnp.maximum(m, tab[l, pl.ds(k * 16, 16)])
    segmax_v[pl.ds(k * 16, 16)] = m
    return 0
  lax.fori_loop(0, S // 16, mfold, 0)

  @pl.when(cid == 0)
  def _():
    pltpu.sync_copy(segmax_v, partm_hbm.at[sid])

  def wbody(i, _):
    g = gate_v[pl.ds(i * 16, 16)]
    sg = seg_v[pl.ds(i * 16, 16)]
    mx = plsc.load_gather(segmax_v, [sg])
    row = base + i * 16 + li
    val = jnp.where(row < N, jnp.exp(g - mx), 0.0)
    wv[pl.ds(i * 16, 16)] = val

    @pl.when(lax.rem(i, 2) == cid)
    def _():
      plsc.addupdate_scatter(dtab, [li, sg], val)
    return 0
  lax.fori_loop(0, VECS, wbody, 0)

  def dfold(k, _):
    m = dtab[0, pl.ds(k * 16, 16)]
    for l in range(1, 16):
      m = m + dtab[l, pl.ds(k * 16, 16)]
    dsum[pl.ds(k * 16, 16)] = m
    return 0
  lax.fori_loop(0, S // 16, dfold, 0)
  pltpu.sync_copy(dsum, partd_hbm.at[cid, sid])

  cols = [c * 16 + lax.iota(jnp.int32, 16) for c in range(DH // 16)]

  def process(j, xb):
    @plsc.parallel_loop(0, SUB // 2, unroll=2)
    def _(rr):
      r = rr * 2
      sp0 = jnp.broadcast_to(j * SUB + r, (16,)).astype(jnp.int32)
      sp1 = jnp.broadcast_to(j * SUB + r + 1, (16,)).astype(jnp.int32)
      w0 = plsc.load_gather(wv, [sp0])
      s0 = plsc.load_gather(seg_v, [sp0])
      w1 = plsc.load_gather(wv, [sp1])
      s1 = plsc.load_gather(seg_v, [sp1])
      for c in range(DH // 16):
        va, vb = plsc.unpack(xb[pl.ds(r * DH + c * 16, 32)],
                             format=plsc.PackFormat.INTERLEAVED)
        plsc.addupdate_scatter(acc, [s0, cols[c]], va * w0)
        plsc.addupdate_scatter(acc, [s1, cols[c]], vb * w1)

  def jbody(jj, _):
    j = jj * 4
    for b in range(4):
      drain(xbufs[b], gsems[b])
      process(j + b, xbufs[b])

      @pl.when(j + b + 4 < NSUB)
      def _():
        gather(j + b + 4, xbufs[b], gsems[b])
    return 0

  lax.fori_loop(0, NSUB // 4, jbody, 0)

  pltpu.sync_copy(acc, part_hbm.at[cid, sid])


def _pool(xl, xr, gate16, batch_i32):
  mesh = plsc.VectorSubcoreMesh(core_axis_name="c", subcore_axis_name="s")
  f = pl.kernel(
      _pool_kernel,
      out_type=[
          jax.ShapeDtypeStruct((2, NR, S, DH), jnp.float32),
          jax.ShapeDtypeStruct((2, NR, S), jnp.float32),
          jax.ShapeDtypeStruct((NR, S), jnp.float32),
      ],
      mesh=mesh,
      compiler_params=pltpu.CompilerParams(needs_layout_passes=False),
      scratch_types=[
          pltpu.VMEM((CHUNK,), jnp.float32),
          pltpu.VMEM((CHUNK,), jnp.int32),
          pltpu.VMEM((CHUNK,), jnp.float32),
          pltpu.VMEM((S,), jnp.float32),
          pltpu.VMEM((16, S), jnp.float32),
          pltpu.VMEM((16, S), jnp.float32),
          pltpu.VMEM((S,), jnp.float32),
          pltpu.VMEM((SUB * DH,), jnp.bfloat16),
          pltpu.VMEM((SUB * DH,), jnp.bfloat16),
          pltpu.VMEM((SUB * DH,), jnp.bfloat16),
          pltpu.VMEM((SUB * DH,), jnp.bfloat16),
          pltpu.VMEM((S, DH), jnp.float32),
          pltpu.SemaphoreType.DMA,
          pltpu.SemaphoreType.DMA,
          pltpu.SemaphoreType.DMA,
          pltpu.SemaphoreType.DMA,
      ],
  )
  return f(xl, xr, gate16, batch_i32)


def _combine_kernel(p_ref, d_ref, m_ref, o_ref):
  pm = m_ref[...]
  m = jnp.max(pm, axis=0, keepdims=True)
  scale = jnp.exp(pm - m)
  dn = jnp.sum(scale * (d_ref[0] + d_ref[1]), axis=0)
  inv = 1.0 / (dn + 1e-16)
  o_ref[:, 0:DH] = jnp.sum(scale[:, :, None] * p_ref[0], axis=0) * inv[:, None]
  o_ref[:, DH:D] = jnp.sum(scale[:, :, None] * p_ref[1], axis=0) * inv[:, None]


def _combine(part, partd, partm):
  return pl.pallas_call(
      _combine_kernel,
      out_shape=jax.ShapeDtypeStruct((S, D), jnp.float32),
  )(part, partd, partm)


@jax.jit
def kernel(x, batch, W1, b1, W2, b2):
  batch_i32 = batch.astype(jnp.int32)
  gate, xl, xr = _gate(x, W1, b1, W2, b2)
  part, partd, partm = _pool(xl.reshape(NPAD * DH), xr.reshape(NPAD * DH),
                             gate.reshape(NR, CHUNK), batch_i32)
  return _combine(part, partd, partm)


# speedup vs baseline: 1.0777x; 1.0777x over previous
import functools

import jax
import jax.numpy as jnp
from jax import lax
from jax.experimental import pallas as pl
from jax.experimental.pallas import tpu as pltpu
from jax.experimental.pallas import tpu_sc as plsc

N = 50000
D = 256
S = 512
NEG = -1e38

NR = 16
CHUNK = 3136
NPAD = NR * CHUNK
SUB = 112
NSUB = CHUNK // SUB
VECS = CHUNK // 16
DH = D // 2

BCH = NPAD // 32
BVECS = BCH // 16


BLK = 3136
NBLK = NPAD // BLK


def _gate_kernel(x_ref, w1_ref, b1_ref, w2_ref, b2_ref, gate_ref,
                 xl_ref, xr_ref):
  b = pl.program_id(0)
  x = x_ref[...]
  h = jnp.maximum(x @ w1_ref[...] + b1_ref[...], 0.0)
  gate = h @ w2_ref[...] + b2_ref[...]
  row = b * BLK + lax.broadcasted_iota(jnp.int32, (BLK, 1), 0)
  valid = row < N
  gate_ref[...] = jnp.where(valid, gate, NEG)
  xz = jnp.where(valid, x, 0.0).astype(jnp.bfloat16)
  xl_ref[...] = xz[:, 0:DH]
  xr_ref[...] = xz[:, DH:D]


def _gate(x, W1, b1, W2, b2):
  return pl.pallas_call(
      _gate_kernel,
      grid=(NBLK,),
      in_specs=[
          pl.BlockSpec((BLK, D), lambda b: (b, 0)),
          pl.BlockSpec((D, D), lambda b: (0, 0)),
          pl.BlockSpec((1, D), lambda b: (0, 0)),
          pl.BlockSpec((D, 1), lambda b: (0, 0)),
          pl.BlockSpec((1, 1), lambda b: (0, 0)),
      ],
      out_specs=[
          pl.BlockSpec((BLK, 1), lambda b: (b, 0)),
          pl.BlockSpec((BLK, DH), lambda b: (b, 0)),
          pl.BlockSpec((BLK, DH), lambda b: (b, 0)),
      ],
      out_shape=[
          jax.ShapeDtypeStruct((NPAD, 1), jnp.float32),
          jax.ShapeDtypeStruct((NPAD, DH), jnp.bfloat16),
          jax.ShapeDtypeStruct((NPAD, DH), jnp.bfloat16),
      ],
  )(x, W1, b1.reshape(1, D), W2, b2.reshape(1, 1))


def _pool_kernel(xl_hbm, xr_hbm, gate_hbm, seg_hbm,
                 part_hbm, partd_hbm, partm_hbm,
                 gate_v, seg_v, wv, segmax_v, tab, dtab, dsum,
                 xbuf0, xbuf1, acc, gsem0, gsem1):
  cid = lax.axis_index("c")
  sid = lax.axis_index("s")
  base = sid * CHUNK
  li = lax.iota(jnp.int32, 16)

  def gather(j, xb, sem):
    off = (base + j * SUB) * DH

    @pl.when(cid == 0)
    def _():
      pltpu.async_copy(xl_hbm.at[pl.ds(off, SUB * DH)], xb, sem)

    @pl.when(cid == 1)
    def _():
      pltpu.async_copy(xr_hbm.at[pl.ds(off, SUB * DH)], xb, sem)

  def drain(xb, sem):
    pltpu.make_async_copy(xl_hbm.at[pl.ds(0, SUB * DH)], xb, sem).wait()

  gather(0, xbuf0, gsem0)
  gather(1, xbuf1, gsem1)

  @plsc.parallel_loop(0, S, unroll=4)
  def _(r):
    for c in range(DH // 16):
      acc[r, pl.ds(c * 16, 16)] = jnp.zeros((16,), jnp.float32)

  @plsc.parallel_loop(0, 16, unroll=2)
  def _(l):
    for k in range(S // 16):
      dtab[l, pl.ds(k * 16, 16)] = jnp.zeros((16,), jnp.float32)
      tab[l, pl.ds(k * 16, 16)] = jnp.full((16,), NEG, jnp.float32)

  pltpu.sync_copy(gate_hbm.at[sid], gate_v)

  tail = N - (NR - 1) * CHUNK

  @pl.when(sid < NR - 1)
  def _():
    pltpu.sync_copy(seg_hbm.at[pl.ds(base, CHUNK)], seg_v)

  @pl.when(sid == NR - 1)
  def _():
    pltpu.sync_copy(seg_hbm.at[pl.ds(base, tail)], seg_v.at[pl.ds(0, tail)])
    for t in range((CHUNK - tail) // 16):
      seg_v[pl.ds(tail + t * 16, 16)] = jnp.zeros((16,), jnp.int32)

  def maxbody(i, _):
    g = gate_v[pl.ds(i * 16, 16)]
    sg = seg_v[pl.ds(i * 16, 16)]
    cur = plsc.load_gather(tab, [li, sg])
    plsc.store_scatter(tab, [li, sg], jnp.maximum(cur, g))
    return 0
  lax.fori_loop(0, VECS, maxbody, 0)

  def mfold(k, _):
    m = tab[0, pl.ds(k * 16, 16)]
    for l in range(1, 16):
      m = jnp.maximum(m, tab[l, pl.ds(k * 16, 16)])
    segmax_v[pl.ds(k * 16, 16)] = m
    return 0
  lax.fori_loop(0, S // 16, mfold, 0)

  @pl.when(cid == 0)
  def _():
    pltpu.sync_copy(segmax_v, partm_hbm.at[sid])

  def wbody(i, _):
    g = gate_v[pl.ds(i * 16, 16)]
    sg = seg_v[pl.ds(i * 16, 16)]
    mx = plsc.load_gather(segmax_v, [sg])
    row = base + i * 16 + li
    val = jnp.where(row < N, jnp.exp(g - mx), 0.0)
    wv[pl.ds(i * 16, 16)] = val

    @pl.when(lax.rem(i, 2) == cid)
    def _():
      plsc.addupdate_scatter(dtab, [li, sg], val)
    return 0
  lax.fori_loop(0, VECS, wbody, 0)

  def dfold(k, _):
    m = dtab[0, pl.ds(k * 16, 16)]
    for l in range(1, 16):
      m = m + dtab[l, pl.ds(k * 16, 16)]
    dsum[pl.ds(k * 16, 16)] = m
    return 0
  lax.fori_loop(0, S // 16, dfold, 0)
  pltpu.sync_copy(dsum, partd_hbm.at[cid, sid])

  cols = [c * 16 + lax.iota(jnp.int32, 16) for c in range(DH // 16)]

  def process(j, xb):
    @plsc.parallel_loop(0, SUB // 2, unroll=1)
    def _(rr):
      r = rr * 2
      sp0 = jnp.broadcast_to(j * SUB + r, (16,)).astype(jnp.int32)
      sp1 = jnp.broadcast_to(j * SUB + r + 1, (16,)).astype(jnp.int32)
      w0 = plsc.load_gather(wv, [sp0])
      s0 = plsc.load_gather(seg_v, [sp0])
      w1 = plsc.load_gather(wv, [sp1])
      s1 = plsc.load_gather(seg_v, [sp1])
      for c in range(DH // 16):
        va, vb = plsc.unpack(xb[pl.ds(r * DH + c * 16, 32)],
                             format=plsc.PackFormat.INTERLEAVED)
        plsc.addupdate_scatter(acc, [s0, cols[c]], va * w0)
        plsc.addupdate_scatter(acc, [s1, cols[c]], vb * w1)

  def jbody(jj, _):
    j = jj * 2
    drain(xbuf0, gsem0)
    process(j, xbuf0)

    @pl.when(j + 2 < NSUB)
    def _():
      gather(j + 2, xbuf0, gsem0)

    drain(xbuf1, gsem1)
    process(j + 1, xbuf1)

    @pl.when(j + 3 < NSUB)
    def _():
      gather(j + 3, xbuf1, gsem1)
    return 0

  lax.fori_loop(0, NSUB // 2, jbody, 0)

  pltpu.sync_copy(acc, part_hbm.at[cid, sid])


def _pool(xl, xr, gate16, batch_i32):
  mesh = plsc.VectorSubcoreMesh(core_axis_name="c", subcore_axis_name="s")
  f = pl.kernel(
      _pool_kernel,
      out_type=[
          jax.ShapeDtypeStruct((2, NR, S, DH), jnp.float32),
          jax.ShapeDtypeStruct((2, NR, S), jnp.float32),
          jax.ShapeDtypeStruct((NR, S), jnp.float32),
      ],
      mesh=mesh,
      compiler_params=pltpu.CompilerParams(needs_layout_passes=False),
      scratch_types=[
          pltpu.VMEM((CHUNK,), jnp.float32),
          pltpu.VMEM((CHUNK,), jnp.int32),
          pltpu.VMEM((CHUNK,), jnp.float32),
          pltpu.VMEM((S,), jnp.float32),
          pltpu.VMEM((16, S), jnp.float32),
          pltpu.VMEM((16, S), jnp.float32),
          pltpu.VMEM((S,), jnp.float32),
          pltpu.VMEM((SUB * DH,), jnp.bfloat16),
          pltpu.VMEM((SUB * DH,), jnp.bfloat16),
          pltpu.VMEM((S, DH), jnp.float32),
          pltpu.SemaphoreType.DMA,
          pltpu.SemaphoreType.DMA,
      ],
  )
  return f(xl, xr, gate16, batch_i32)


def _combine_kernel(p_ref, d_ref, m_ref, o_ref):
  pm = m_ref[...]
  m = jnp.max(pm, axis=0, keepdims=True)
  scale = jnp.exp(pm - m)
  dn = jnp.sum(scale * (d_ref[0] + d_ref[1]), axis=0)
  inv = 1.0 / (dn + 1e-16)
  o_ref[:, 0:DH] = jnp.sum(scale[:, :, None] * p_ref[0], axis=0) * inv[:, None]
  o_ref[:, DH:D] = jnp.sum(scale[:, :, None] * p_ref[1], axis=0) * inv[:, None]


def _combine(part, partd, partm):
  return pl.pallas_call(
      _combine_kernel,
      out_shape=jax.ShapeDtypeStruct((S, D), jnp.float32),
  )(part, partd, partm)


@jax.jit
def kernel(x, batch, W1, b1, W2, b2):
  batch_i32 = batch.astype(jnp.int32)
  gate, xl, xr = _gate(x, W1, b1, W2, b2)
  part, partd, partm = _pool(xl.reshape(NPAD * DH), xr.reshape(NPAD * DH),
                             gate.reshape(NR, CHUNK), batch_i32)
  return _combine(part, partd, partm)


# speedup vs baseline: 1.1107x; 1.0307x over previous
import functools

import jax
import jax.numpy as jnp
from jax import lax
from jax.experimental import pallas as pl
from jax.experimental.pallas import tpu as pltpu
from jax.experimental.pallas import tpu_sc as plsc

N = 50000
D = 256
S = 512
NEG = -1e38

NR = 16
CHUNK = 3136
NPAD = NR * CHUNK
SUB = 112
NSUB = CHUNK // SUB
VECS = CHUNK // 16
DH = D // 2

BCH = NPAD // 32
BVECS = BCH // 16


BLK = 6272
NBLK = NPAD // BLK


def _gate_kernel(x_ref, w1_ref, b1_ref, w2_ref, b2_ref, gate_ref,
                 xl_ref, xr_ref):
  b = pl.program_id(0)
  x = x_ref[...]
  h = jnp.maximum(x @ w1_ref[...] + b1_ref[...], 0.0)
  gate = h @ w2_ref[...] + b2_ref[...]
  row = b * BLK + lax.broadcasted_iota(jnp.int32, (BLK, 1), 0)
  valid = row < N
  gate_ref[...] = jnp.where(valid, gate, NEG)
  xz = jnp.where(valid, x, 0.0).astype(jnp.bfloat16)
  xl_ref[...] = xz[:, 0:DH]
  xr_ref[...] = xz[:, DH:D]


def _gate(x, W1, b1, W2, b2):
  return pl.pallas_call(
      _gate_kernel,
      grid=(NBLK,),
      in_specs=[
          pl.BlockSpec((BLK, D), lambda b: (b, 0)),
          pl.BlockSpec((D, D), lambda b: (0, 0)),
          pl.BlockSpec((1, D), lambda b: (0, 0)),
          pl.BlockSpec((D, 1), lambda b: (0, 0)),
          pl.BlockSpec((1, 1), lambda b: (0, 0)),
      ],
      out_specs=[
          pl.BlockSpec((BLK, 1), lambda b: (b, 0)),
          pl.BlockSpec((BLK, DH), lambda b: (b, 0)),
          pl.BlockSpec((BLK, DH), lambda b: (b, 0)),
      ],
      out_shape=[
          jax.ShapeDtypeStruct((NPAD, 1), jnp.float32),
          jax.ShapeDtypeStruct((NPAD, DH), jnp.bfloat16),
          jax.ShapeDtypeStruct((NPAD, DH), jnp.bfloat16),
      ],
  )(x, W1, b1.reshape(1, D), W2, b2.reshape(1, 1))


def _pool_kernel(xl_hbm, xr_hbm, gate_hbm, seg_hbm,
                 part_hbm, partd_hbm, partm_hbm,
                 gate_v, seg_v, wv, segmax_v, tab, dtab, dsum,
                 xbuf0, xbuf1, acc, gsem0, gsem1):
  cid = lax.axis_index("c")
  sid = lax.axis_index("s")
  base = sid * CHUNK
  li = lax.iota(jnp.int32, 16)

  def gather(j, xb, sem):
    off = (base + j * SUB) * DH

    @pl.when(cid == 0)
    def _():
      pltpu.async_copy(xl_hbm.at[pl.ds(off, SUB * DH)], xb, sem)

    @pl.when(cid == 1)
    def _():
      pltpu.async_copy(xr_hbm.at[pl.ds(off, SUB * DH)], xb, sem)

  def drain(xb, sem):
    pltpu.make_async_copy(xl_hbm.at[pl.ds(0, SUB * DH)], xb, sem).wait()

  gather(0, xbuf0, gsem0)
  gather(1, xbuf1, gsem1)

  @plsc.parallel_loop(0, S, unroll=4)
  def _(r):
    for c in range(DH // 16):
      acc[r, pl.ds(c * 16, 16)] = jnp.zeros((16,), jnp.float32)

  @plsc.parallel_loop(0, 16, unroll=2)
  def _(l):
    for k in range(S // 16):
      dtab[l, pl.ds(k * 16, 16)] = jnp.zeros((16,), jnp.float32)
      tab[l, pl.ds(k * 16, 16)] = jnp.full((16,), NEG, jnp.float32)

  pltpu.sync_copy(gate_hbm.at[sid], gate_v)

  tail = N - (NR - 1) * CHUNK

  @pl.when(sid < NR - 1)
  def _():
    pltpu.sync_copy(seg_hbm.at[pl.ds(base, CHUNK)], seg_v)

  @pl.when(sid == NR - 1)
  def _():
    pltpu.sync_copy(seg_hbm.at[pl.ds(base, tail)], seg_v.at[pl.ds(0, tail)])
    for t in range((CHUNK - tail) // 16):
      seg_v[pl.ds(tail + t * 16, 16)] = jnp.zeros((16,), jnp.int32)

  def maxbody(i, _):
    g = gate_v[pl.ds(i * 16, 16)]
    sg = seg_v[pl.ds(i * 16, 16)]
    cur = plsc.load_gather(tab, [li, sg])
    plsc.store_scatter(tab, [li, sg], jnp.maximum(cur, g))
    return 0
  lax.fori_loop(0, VECS, maxbody, 0)

  def mfold(k, _):
    m = tab[0, pl.ds(k * 16, 16)]
    for l in range(1, 16):
      m = jnp.maximum(m, tab[l, pl.ds(k * 16, 16)])
    segmax_v[pl.ds(k * 16, 16)] = m
    return 0
  lax.fori_loop(0, S // 16, mfold, 0)

  @pl.when(cid == 0)
  def _():
    pltpu.sync_copy(segmax_v, partm_hbm.at[sid])

  def wbody(i, _):
    g = gate_v[pl.ds(i * 16, 16)]
    sg = seg_v[pl.ds(i * 16, 16)]
    mx = plsc.load_gather(segmax_v, [sg])
    row = base + i * 16 + li
    val = jnp.where(row < N, jnp.exp(g - mx), 0.0)
    wv[pl.ds(i * 16, 16)] = val

    @pl.when(lax.rem(i, 2) == cid)
    def _():
      plsc.addupdate_scatter(dtab, [li, sg], val)
    return 0
  lax.fori_loop(0, VECS, wbody, 0)

  def dfold(k, _):
    m = dtab[0, pl.ds(k * 16, 16)]
    for l in range(1, 16):
      m = m + dtab[l, pl.ds(k * 16, 16)]
    dsum[pl.ds(k * 16, 16)] = m
    return 0
  lax.fori_loop(0, S // 16, dfold, 0)
  pltpu.sync_copy(dsum, partd_hbm.at[cid, sid])

  cols = [c * 16 + lax.iota(jnp.int32, 16) for c in range(DH // 16)]

  def process(j, xb):
    @plsc.parallel_loop(0, SUB // 2, unroll=1)
    def _(rr):
      r = rr * 2
      sp0 = jnp.broadcast_to(j * SUB + r, (16,)).astype(jnp.int32)
      sp1 = jnp.broadcast_to(j * SUB + r + 1, (16,)).astype(jnp.int32)
      w0 = plsc.load_gather(wv, [sp0])
      s0 = plsc.load_gather(seg_v, [sp0])
      w1 = plsc.load_gather(wv, [sp1])
      s1 = plsc.load_gather(seg_v, [sp1])
      for c in range(DH // 16):
        va, vb = plsc.unpack(xb[pl.ds(r * DH + c * 16, 32)],
                             format=plsc.PackFormat.INTERLEAVED)
        plsc.addupdate_scatter(acc, [s0, cols[c]], va * w0)
        plsc.addupdate_scatter(acc, [s1, cols[c]], vb * w1)

  def jbody(jj, _):
    j = jj * 2
    drain(xbuf0, gsem0)
    process(j, xbuf0)

    @pl.when(j + 2 < NSUB)
    def _():
      gather(j + 2, xbuf0, gsem0)

    drain(xbuf1, gsem1)
    process(j + 1, xbuf1)

    @pl.when(j + 3 < NSUB)
    def _():
      gather(j + 3, xbuf1, gsem1)
    return 0

  lax.fori_loop(0, NSUB // 2, jbody, 0)

  pltpu.sync_copy(acc, part_hbm.at[cid, sid])


def _pool(xl, xr, gate16, batch_i32):
  mesh = plsc.VectorSubcoreMesh(core_axis_name="c", subcore_axis_name="s")
  f = pl.kernel(
      _pool_kernel,
      out_type=[
          jax.ShapeDtypeStruct((2, NR, S, DH), jnp.float32),
          jax.ShapeDtypeStruct((2, NR, S), jnp.float32),
          jax.ShapeDtypeStruct((NR, S), jnp.float32),
      ],
      mesh=mesh,
      compiler_params=pltpu.CompilerParams(needs_layout_passes=False),
      scratch_types=[
          pltpu.VMEM((CHUNK,), jnp.float32),
          pltpu.VMEM((CHUNK,), jnp.int32),
          pltpu.VMEM((CHUNK,), jnp.float32),
          pltpu.VMEM((S,), jnp.float32),
          pltpu.VMEM((16, S), jnp.float32),
          pltpu.VMEM((16, S), jnp.float32),
          pltpu.VMEM((S,), jnp.float32),
          pltpu.VMEM((SUB * DH,), jnp.bfloat16),
          pltpu.VMEM((SUB * DH,), jnp.bfloat16),
          pltpu.VMEM((S, DH), jnp.float32),
          pltpu.SemaphoreType.DMA,
          pltpu.SemaphoreType.DMA,
      ],
  )
  return f(xl, xr, gate16, batch_i32)


def _combine_kernel(p_ref, d_ref, m_ref, o_ref):
  pm = m_ref[...]
  m = jnp.max(pm, axis=0, keepdims=True)
  scale = jnp.exp(pm - m)
  dn = jnp.sum(scale * (d_ref[0] + d_ref[1]), axis=0)
  inv = 1.0 / (dn + 1e-16)
  o_ref[:, 0:DH] = jnp.sum(scale[:, :, None] * p_ref[0], axis=0) * inv[:, None]
  o_ref[:, DH:D] = jnp.sum(scale[:, :, None] * p_ref[1], axis=0) * inv[:, None]


def _combine(part, partd, partm):
  return pl.pallas_call(
      _combine_kernel,
      out_shape=jax.ShapeDtypeStruct((S, D), jnp.float32),
  )(part, partd, partm)


@jax.jit
def kernel(x, batch, W1, b1, W2, b2):
  batch_i32 = batch.astype(jnp.int32)
  gate, xl, xr = _gate(x, W1, b1, W2, b2)
  part, partd, partm = _pool(xl.reshape(NPAD * DH), xr.reshape(NPAD * DH),
                             gate.reshape(NR, CHUNK), batch_i32)
  return _combine(part, partd, partm)


# speedup vs baseline: 1.1127x; 1.0018x over previous
import functools

import jax
import jax.numpy as jnp
from jax import lax
from jax.experimental import pallas as pl
from jax.experimental.pallas import tpu as pltpu
from jax.experimental.pallas import tpu_sc as plsc

N = 50000
D = 256
S = 512
NEG = -1e38

NR = 16
CHUNK = 3136
NPAD = NR * CHUNK
SUB = 112
NSUB = CHUNK // SUB
VECS = CHUNK // 16
DH = D // 2

BCH = NPAD // 32
BVECS = BCH // 16


BLK = 12544
NBLK = NPAD // BLK


def _gate_kernel(x_ref, w1_ref, b1_ref, w2_ref, b2_ref, gate_ref,
                 xl_ref, xr_ref):
  b = pl.program_id(0)
  x = x_ref[...]
  h = jnp.maximum(x @ w1_ref[...] + b1_ref[...], 0.0)
  gate = h @ w2_ref[...] + b2_ref[...]
  row = b * BLK + lax.broadcasted_iota(jnp.int32, (BLK, 1), 0)
  valid = row < N
  gate_ref[...] = jnp.where(valid, gate, NEG)
  xz = jnp.where(valid, x, 0.0).astype(jnp.bfloat16)
  xl_ref[...] = xz[:, 0:DH]
  xr_ref[...] = xz[:, DH:D]


def _gate(x, W1, b1, W2, b2):
  return pl.pallas_call(
      _gate_kernel,
      grid=(NBLK,),
      in_specs=[
          pl.BlockSpec((BLK, D), lambda b: (b, 0)),
          pl.BlockSpec((D, D), lambda b: (0, 0)),
          pl.BlockSpec((1, D), lambda b: (0, 0)),
          pl.BlockSpec((D, 1), lambda b: (0, 0)),
          pl.BlockSpec((1, 1), lambda b: (0, 0)),
      ],
      out_specs=[
          pl.BlockSpec((BLK, 1), lambda b: (b, 0)),
          pl.BlockSpec((BLK, DH), lambda b: (b, 0)),
          pl.BlockSpec((BLK, DH), lambda b: (b, 0)),
      ],
      out_shape=[
          jax.ShapeDtypeStruct((NPAD, 1), jnp.float32),
          jax.ShapeDtypeStruct((NPAD, DH), jnp.bfloat16),
          jax.ShapeDtypeStruct((NPAD, DH), jnp.bfloat16),
      ],
  )(x, W1, b1.reshape(1, D), W2, b2.reshape(1, 1))


def _pool_kernel(xl_hbm, xr_hbm, gate_hbm, seg_hbm,
                 part_hbm, partd_hbm, partm_hbm,
                 gate_v, seg_v, wv, segmax_v, tab, dtab, dsum,
                 xbuf0, xbuf1, acc, gsem0, gsem1):
  cid = lax.axis_index("c")
  sid = lax.axis_index("s")
  base = sid * CHUNK
  li = lax.iota(jnp.int32, 16)

  def gather(j, xb, sem):
    off = (base + j * SUB) * DH

    @pl.when(cid == 0)
    def _():
      pltpu.async_copy(xl_hbm.at[pl.ds(off, SUB * DH)], xb, sem)

    @pl.when(cid == 1)
    def _():
      pltpu.async_copy(xr_hbm.at[pl.ds(off, SUB * DH)], xb, sem)

  def drain(xb, sem):
    pltpu.make_async_copy(xl_hbm.at[pl.ds(0, SUB * DH)], xb, sem).wait()

  gather(0, xbuf0, gsem0)
  gather(1, xbuf1, gsem1)

  @plsc.parallel_loop(0, S, unroll=4)
  def _(r):
    for c in range(DH // 16):
      acc[r, pl.ds(c * 16, 16)] = jnp.zeros((16,), jnp.float32)

  @plsc.parallel_loop(0, 16, unroll=2)
  def _(l):
    for k in range(S // 16):
      dtab[l, pl.ds(k * 16, 16)] = jnp.zeros((16,), jnp.float32)
      tab[l, pl.ds(k * 16, 16)] = jnp.full((16,), NEG, jnp.float32)

  pltpu.sync_copy(gate_hbm.at[sid], gate_v)

  tail = N - (NR - 1) * CHUNK

  @pl.when(sid < NR - 1)
  def _():
    pltpu.sync_copy(seg_hbm.at[pl.ds(base, CHUNK)], seg_v)

  @pl.when(sid == NR - 1)
  def _():
    pltpu.sync_copy(seg_hbm.at[pl.ds(base, tail)], seg_v.at[pl.ds(0, tail)])
    for t in range((CHUNK - tail) // 16):
      seg_v[pl.ds(tail + t * 16, 16)] = jnp.zeros((16,), jnp.int32)

  def maxbody(i, _):
    g = gate_v[pl.ds(i * 16, 16)]
    sg = seg_v[pl.ds(i * 16, 16)]
    cur = plsc.load_gather(tab, [li, sg])
    plsc.store_scatter(tab, [li, sg], jnp.maximum(cur, g))
    return 0
  lax.fori_loop(0, VECS, maxbody, 0)

  def mfold(k, _):
    m = tab[0, pl.ds(k * 16, 16)]
    for l in range(1, 16):
      m = jnp.maximum(m, tab[l, pl.ds(k * 16, 16)])
    segmax_v[pl.ds(k * 16, 16)] = m
    return 0
  lax.fori_loop(0, S // 16, mfold, 0)

  @pl.when(cid == 0)
  def _():
    pltpu.sync_copy(segmax_v, partm_hbm.at[sid])

  def wbody(i, _):
    g = gate_v[pl.ds(i * 16, 16)]
    sg = seg_v[pl.ds(i * 16, 16)]
    mx = plsc.load_gather(segmax_v, [sg])
    row = base + i * 16 + li
    val = jnp.where(row < N, jnp.exp(g - mx), 0.0)
    wv[pl.ds(i * 16, 16)] = val

    @pl.when(lax.rem(i, 2) == cid)
    def _():
      plsc.addupdate_scatter(dtab, [li, sg], val)
    return 0
  lax.fori_loop(0, VECS, wbody, 0)

  def dfold(k, _):
    m = dtab[0, pl.ds(k * 16, 16)]
    for l in range(1, 16):
      m = m + dtab[l, pl.ds(k * 16, 16)]
    dsum[pl.ds(k * 16, 16)] = m
    return 0
  lax.fori_loop(0, S // 16, dfold, 0)
  pltpu.sync_copy(dsum, partd_hbm.at[cid, sid])

  cols = [c * 16 + lax.iota(jnp.int32, 16) for c in range(DH // 16)]

  def process(j, xb):
    @plsc.parallel_loop(0, SUB // 2, unroll=1)
    def _(rr):
      r = rr * 2
      sp0 = jnp.broadcast_to(j * SUB + r, (16,)).astype(jnp.int32)
      sp1 = jnp.broadcast_to(j * SUB + r + 1, (16,)).astype(jnp.int32)
      w0 = plsc.load_gather(wv, [sp0])
      s0 = plsc.load_gather(seg_v, [sp0])
      w1 = plsc.load_gather(wv, [sp1])
      s1 = plsc.load_gather(seg_v, [sp1])
      for c in range(DH // 16):
        va, vb = plsc.unpack(xb[pl.ds(r * DH + c * 16, 32)],
                             format=plsc.PackFormat.INTERLEAVED)
        plsc.addupdate_scatter(acc, [s0, cols[c]], va * w0)
        plsc.addupdate_scatter(acc, [s1, cols[c]], vb * w1)

  def jbody(jj, _):
    j = jj * 2
    drain(xbuf0, gsem0)
    process(j, xbuf0)

    @pl.when(j + 2 < NSUB)
    def _():
      gather(j + 2, xbuf0, gsem0)

    drain(xbuf1, gsem1)
    process(j + 1, xbuf1)

    @pl.when(j + 3 < NSUB)
    def _():
      gather(j + 3, xbuf1, gsem1)
    return 0

  lax.fori_loop(0, NSUB // 2, jbody, 0)

  pltpu.sync_copy(acc, part_hbm.at[cid, sid])


def _pool(xl, xr, gate16, batch_i32):
  mesh = plsc.VectorSubcoreMesh(core_axis_name="c", subcore_axis_name="s")
  f = pl.kernel(
      _pool_kernel,
      out_type=[
          jax.ShapeDtypeStruct((2, NR, S, DH), jnp.float32),
          jax.ShapeDtypeStruct((2, NR, S), jnp.float32),
          jax.ShapeDtypeStruct((NR, S), jnp.float32),
      ],
      mesh=mesh,
      compiler_params=pltpu.CompilerParams(needs_layout_passes=False),
      scratch_types=[
          pltpu.VMEM((CHUNK,), jnp.float32),
          pltpu.VMEM((CHUNK,), jnp.int32),
          pltpu.VMEM((CHUNK,), jnp.float32),
          pltpu.VMEM((S,), jnp.float32),
          pltpu.VMEM((16, S), jnp.float32),
          pltpu.VMEM((16, S), jnp.float32),
          pltpu.VMEM((S,), jnp.float32),
          pltpu.VMEM((SUB * DH,), jnp.bfloat16),
          pltpu.VMEM((SUB * DH,), jnp.bfloat16),
          pltpu.VMEM((S, DH), jnp.float32),
          pltpu.SemaphoreType.DMA,
          pltpu.SemaphoreType.DMA,
      ],
  )
  return f(xl, xr, gate16, batch_i32)


def _combine_kernel(p_ref, d_ref, m_ref, o_ref):
  pm = m_ref[...]
  m = jnp.max(pm, axis=0, keepdims=True)
  scale = jnp.exp(pm - m)
  dn = jnp.sum(scale * (d_ref[0] + d_ref[1]), axis=0)
  inv = 1.0 / (dn + 1e-16)
  o_ref[:, 0:DH] = jnp.sum(scale[:, :, None] * p_ref[0], axis=0) * inv[:, None]
  o_ref[:, DH:D] = jnp.sum(scale[:, :, None] * p_ref[1], axis=0) * inv[:, None]


def _combine(part, partd, partm):
  return pl.pallas_call(
      _combine_kernel,
      out_shape=jax.ShapeDtypeStruct((S, D), jnp.float32),
  )(part, partd, partm)


@jax.jit
def kernel(x, batch, W1, b1, W2, b2):
  batch_i32 = batch.astype(jnp.int32)
  gate, xl, xr = _gate(x, W1, b1, W2, b2)
  part, partd, partm = _pool(xl.reshape(NPAD * DH), xr.reshape(NPAD * DH),
                             gate.reshape(NR, CHUNK), batch_i32)
  return _combine(part, partd, partm)


# speedup vs baseline: 1.1132x; 1.0004x over previous
import jax
import jax.numpy as jnp
from jax import lax
from jax.experimental import pallas as pl
from jax.experimental.pallas import tpu as pltpu
from jax.experimental.pallas import tpu_sc as plsc

N = 50000
D = 256
S = 512
NEG = -1e38

NR = 16
CHUNK = 3136
NPAD = NR * CHUNK
SUB = 112
NSUB = CHUNK // SUB
VECS = CHUNK // 16
DH = D // 2

BCH = NPAD // 32
BVECS = BCH // 16


BLK = 12544
NBLK = NPAD // BLK


def _gate_kernel(x_ref, w1_ref, b1_ref, w2_ref, b2_ref, gate_ref,
                 xl_ref, xr_ref):
  b = pl.program_id(0)
  x = x_ref[...]
  h = jnp.maximum(x @ w1_ref[...] + b1_ref[...], 0.0)
  gate = h @ w2_ref[...] + b2_ref[...]
  row = b * BLK + lax.broadcasted_iota(jnp.int32, (BLK, 1), 0)
  valid = row < N
  gate_ref[...] = jnp.where(valid, gate, NEG)
  xz = jnp.where(valid, x, 0.0).astype(jnp.bfloat16)
  xl_ref[...] = xz[:, 0:DH]
  xr_ref[...] = xz[:, DH:D]


def _gate(x, W1, b1, W2, b2):
  return pl.pallas_call(
      _gate_kernel,
      grid=(NBLK,),
      in_specs=[
          pl.BlockSpec((BLK, D), lambda b: (b, 0)),
          pl.BlockSpec((D, D), lambda b: (0, 0)),
          pl.BlockSpec((1, D), lambda b: (0, 0)),
          pl.BlockSpec((D, 1), lambda b: (0, 0)),
          pl.BlockSpec((1, 1), lambda b: (0, 0)),
      ],
      out_specs=[
          pl.BlockSpec((BLK, 1), lambda b: (b, 0)),
          pl.BlockSpec((BLK, DH), lambda b: (b, 0)),
          pl.BlockSpec((BLK, DH), lambda b: (b, 0)),
      ],
      out_shape=[
          jax.ShapeDtypeStruct((NPAD, 1), jnp.float32),
          jax.ShapeDtypeStruct((NPAD, DH), jnp.bfloat16),
          jax.ShapeDtypeStruct((NPAD, DH), jnp.bfloat16),
      ],
  )(x, W1, b1.reshape(1, D), W2, b2.reshape(1, 1))


def _pool_kernel(xl_hbm, xr_hbm, gate_hbm, seg_hbm,
                 part_hbm, partd_hbm, partm_hbm,
                 gate_v, seg_v, wv, segmax_v, tab, dtab, dsum,
                 xbuf0, xbuf1, acc, gsem0, gsem1):
  cid = lax.axis_index("c")
  sid = lax.axis_index("s")
  base = sid * CHUNK
  li = lax.iota(jnp.int32, 16)

  def gather(j, xb, sem):
    off = (base + j * SUB) * DH

    @pl.when(cid == 0)
    def _():
      pltpu.async_copy(xl_hbm.at[pl.ds(off, SUB * DH)], xb, sem)

    @pl.when(cid == 1)
    def _():
      pltpu.async_copy(xr_hbm.at[pl.ds(off, SUB * DH)], xb, sem)

  def drain(xb, sem):
    pltpu.make_async_copy(xl_hbm.at[pl.ds(0, SUB * DH)], xb, sem).wait()

  gather(0, xbuf0, gsem0)
  gather(1, xbuf1, gsem1)

  @plsc.parallel_loop(0, S, unroll=4)
  def _(r):
    for c in range(DH // 16):
      acc[r, pl.ds(c * 16, 16)] = jnp.zeros((16,), jnp.float32)

  @plsc.parallel_loop(0, 16, unroll=2)
  def _(l):
    for k in range(S // 16):
      dtab[l, pl.ds(k * 16, 16)] = jnp.zeros((16,), jnp.float32)
      tab[l, pl.ds(k * 16, 16)] = jnp.full((16,), NEG, jnp.float32)

  pltpu.sync_copy(gate_hbm.at[sid], gate_v)

  tail = N - (NR - 1) * CHUNK

  @pl.when(sid < NR - 1)
  def _():
    pltpu.sync_copy(seg_hbm.at[pl.ds(base, CHUNK)], seg_v)

  @pl.when(sid == NR - 1)
  def _():
    pltpu.sync_copy(seg_hbm.at[pl.ds(base, tail)], seg_v.at[pl.ds(0, tail)])
    for t in range((CHUNK - tail) // 16):
      seg_v[pl.ds(tail + t * 16, 16)] = jnp.zeros((16,), jnp.int32)

  def maxbody(i, _):
    g = gate_v[pl.ds(i * 16, 16)]
    sg = seg_v[pl.ds(i * 16, 16)]
    cur = plsc.load_gather(tab, [li, sg])
    plsc.store_scatter(tab, [li, sg], jnp.maximum(cur, g))
    return 0
  lax.fori_loop(0, VECS, maxbody, 0)

  def mfold(k, _):
    m = tab[0, pl.ds(k * 16, 16)]
    for l in range(1, 16):
      m = jnp.maximum(m, tab[l, pl.ds(k * 16, 16)])
    segmax_v[pl.ds(k * 16, 16)] = m
    return 0
  lax.fori_loop(0, S // 16, mfold, 0)

  @pl.when(cid == 0)
  def _():
    pltpu.sync_copy(segmax_v, partm_hbm.at[sid])

  def wbody(i, _):
    g = gate_v[pl.ds(i * 16, 16)]
    sg = seg_v[pl.ds(i * 16, 16)]
    mx = plsc.load_gather(segmax_v, [sg])
    row = base + i * 16 + li
    val = jnp.where(row < N, jnp.exp(g - mx), 0.0)
    wv[pl.ds(i * 16, 16)] = val

    @pl.when(lax.rem(i, 2) == cid)
    def _():
      plsc.addupdate_scatter(dtab, [li, sg], val)
    return 0
  lax.fori_loop(0, VECS, wbody, 0)

  def dfold(k, _):
    m = dtab[0, pl.ds(k * 16, 16)]
    for l in range(1, 16):
      m = m + dtab[l, pl.ds(k * 16, 16)]
    dsum[pl.ds(k * 16, 16)] = m
    return 0
  lax.fori_loop(0, S // 16, dfold, 0)
  pltpu.sync_copy(dsum, partd_hbm.at[cid, sid])

  cols = [c * 16 + lax.iota(jnp.int32, 16) for c in range(DH // 16)]

  def process(j, xb):
    @plsc.parallel_loop(0, SUB // 2, unroll=1)
    def _(rr):
      r = rr * 2
      sp0 = jnp.broadcast_to(j * SUB + r, (16,)).astype(jnp.int32)
      sp1 = jnp.broadcast_to(j * SUB + r + 1, (16,)).astype(jnp.int32)
      w0 = plsc.load_gather(wv, [sp0])
      s0 = plsc.load_gather(seg_v, [sp0])
      w1 = plsc.load_gather(wv, [sp1])
      s1 = plsc.load_gather(seg_v, [sp1])
      for c in range(DH // 16):
        va, vb = plsc.unpack(xb[pl.ds(r * DH + c * 16, 32)],
                             format=plsc.PackFormat.INTERLEAVED)
        plsc.addupdate_scatter(acc, [s0, cols[c]], va * w0)
        plsc.addupdate_scatter(acc, [s1, cols[c]], vb * w1)

  def jbody(jj, _):
    j = jj * 2
    drain(xbuf0, gsem0)
    process(j, xbuf0)

    @pl.when(j + 2 < NSUB)
    def _():
      gather(j + 2, xbuf0, gsem0)

    drain(xbuf1, gsem1)
    process(j + 1, xbuf1)

    @pl.when(j + 3 < NSUB)
    def _():
      gather(j + 3, xbuf1, gsem1)
    return 0

  lax.fori_loop(0, NSUB // 2, jbody, 0)

  pltpu.sync_copy(acc, part_hbm.at[cid, sid])


def _pool(xl, xr, gate16, batch_i32):
  mesh = plsc.VectorSubcoreMesh(core_axis_name="c", subcore_axis_name="s")
  f = pl.kernel(
      _pool_kernel,
      out_type=[
          jax.ShapeDtypeStruct((2, NR, S, DH), jnp.float32),
          jax.ShapeDtypeStruct((2, NR, S), jnp.float32),
          jax.ShapeDtypeStruct((NR, S), jnp.float32),
      ],
      mesh=mesh,
      compiler_params=pltpu.CompilerParams(needs_layout_passes=False),
      scratch_types=[
          pltpu.VMEM((CHUNK,), jnp.float32),
          pltpu.VMEM((CHUNK,), jnp.int32),
          pltpu.VMEM((CHUNK,), jnp.float32),
          pltpu.VMEM((S,), jnp.float32),
          pltpu.VMEM((16, S), jnp.float32),
          pltpu.VMEM((16, S), jnp.float32),
          pltpu.VMEM((S,), jnp.float32),
          pltpu.VMEM((SUB * DH,), jnp.bfloat16),
          pltpu.VMEM((SUB * DH,), jnp.bfloat16),
          pltpu.VMEM((S, DH), jnp.float32),
          pltpu.SemaphoreType.DMA,
          pltpu.SemaphoreType.DMA,
      ],
  )
  return f(xl, xr, gate16, batch_i32)


def _combine_kernel(p_ref, d_ref, m_ref, o_ref):
  pm = m_ref[...]
  m = jnp.max(pm, axis=0, keepdims=True)
  scale = jnp.exp(pm - m)
  dn = jnp.sum(scale * (d_ref[0] + d_ref[1]), axis=0)
  inv = 1.0 / (dn + 1e-16)
  o_ref[:, 0:DH] = jnp.sum(scale[:, :, None] * p_ref[0], axis=0) * inv[:, None]
  o_ref[:, DH:D] = jnp.sum(scale[:, :, None] * p_ref[1], axis=0) * inv[:, None]


def _combine(part, partd, partm):
  return pl.pallas_call(
      _combine_kernel,
      out_shape=jax.ShapeDtypeStruct((S, D), jnp.float32),
  )(part, partd, partm)


@jax.jit
def kernel(x, batch, W1, b1, W2, b2):
  batch_i32 = batch.astype(jnp.int32)
  gate, xl, xr = _gate(x, W1, b1, W2, b2)
  part, partd, partm = _pool(xl.reshape(NPAD * DH), xr.reshape(NPAD * DH),
                             gate.reshape(NR, CHUNK), batch_i32)
  return _combine(part, partd, partm)
